# baseline (speedup 1.0000x reference)
import jax
import jax.numpy as jnp
from jax import lax
from jax.experimental import pallas as pl
from jax.experimental.pallas import tpu as pltpu
from jax.experimental.pallas import tpu_sc as plsc

_B, _L, _D = 4096, 200, 16
_NC, _NS = 2, 16
_NW = _NC * _NS
_BPW = _B // _NW
_CHUNK_B = 8
_N_CHUNK = _BPW // _CHUNK_B
_ROWS = _CHUNK_B * _L
_PAD_ROWS = _ROWS + 8
_NBLK = 13


_N_MID, _N_UID = 1000000, 100000
_MID_PW, _MID_RB, _MID_NB = 31248, 1008, 31
_UID_PW, _UID_RB, _UID_NB = 3120, 1040, 3
_MID_XTRA, _UID_XTRA = 64, 160


def _tr_body(mid_pl, uid_pl, mid_rm, uid_rm,
             stripe0, outb0, stripe1, outb1, si0, si1, so0, so1):
    wid = lax.axis_index("c") * _NS + lax.axis_index("s")
    stripe = (stripe0, stripe1)
    outb = (outb0, outb1)
    si = (si0, si1)
    so = (so0, so1)

    def fire(p, planar, n, rbase, rb):
        for d in range(16):
            pltpu.async_copy(planar.at[pl.ds(d * n + rbase, rb)],
                             stripe[p].at[pl.ds(d * rb, rb)], si[p])

    def drain(p, planar, n, rbase, rb):
        for d in range(16):
            pltpu.make_async_copy(planar.at[pl.ds(d * n + rbase, rb)],
                                  stripe[p].at[pl.ds(d * rb, rb)],
                                  si[p]).wait()

    def xpose(p, rb, out_ref, rbase, first):
        @pl.when(jnp.logical_not(first))
        def _():
            pltpu.make_async_copy(outb[p].at[pl.ds(0, rb)],
                                  out_ref.at[pl.ds(rbase, rb)], so[p]).wait()

        @pl.loop(0, rb // 16)
        def _g(g):
            iota = lax.iota(jnp.int32, 16)
            rv = iota + g * 16
            for d in range(16):
                vals = stripe[p][pl.ds(d * rb + g * 16, 16)]
                plsc.store_scatter(outb[p], [rv, jnp.full((16,), d,
                                                          jnp.int32)], vals)
        pltpu.async_copy(outb[p].at[pl.ds(0, rb)],
                         out_ref.at[pl.ds(rbase, rb)], so[p])

    def do_table(planar, out_ref, n, pw, rb, nb):
        start = wid * pw
        fire(0, planar, n, start, rb)

        @pl.loop(0, nb, step=2)
        def _blks(b0):
            for ph in range(2):
                p = ph
                b = b0 + ph

                @pl.when(b < nb)
                def _():
                    drain(p, planar, n, start + b * rb, rb)

                    @pl.when(b + 1 < nb)
                    def _():
                        fire(1 - p, planar, n, start + (b + 1) * rb, rb)

                    xpose(p, rb, out_ref, start + b * rb, b < 2)
        for b in (nb - 2, nb - 1):
            if b >= 0:
                p = b % 2
                pltpu.make_async_copy(outb[p].at[pl.ds(0, rb)],
                                      out_ref.at[pl.ds(start + b * rb, rb)],
                                      so[p]).wait()

    do_table(mid_pl, mid_rm, _N_MID, _MID_PW, _MID_RB, _MID_NB)
    do_table(uid_pl, uid_rm, _N_UID, _UID_PW, _UID_RB, _UID_NB)

    @pl.when(wid == _NW - 1)
    def _tail():
        for planar, out_ref, n, tstart, rb in (
                (mid_pl, mid_rm, _N_MID, _MID_PW * _NW, _MID_XTRA),
                (uid_pl, uid_rm, _N_UID, _UID_PW * _NW, _UID_XTRA)):
            fire(0, planar, n, tstart, rb)
            drain(0, planar, n, tstart, rb)

            @pl.loop(0, rb // 16)
            def _g(g):
                iota = lax.iota(jnp.int32, 16)
                rv = iota + g * 16
                for d in range(16):
                    vals = stripe[0][pl.ds(d * rb + g * 16, 16)]
                    plsc.store_scatter(outb[0], [rv, jnp.full(
                        (16,), d, jnp.int32)], vals)
            pltpu.sync_copy(outb[0].at[pl.ds(0, rb)],
                            out_ref.at[pl.ds(tstart, rb)])


_sc_transpose = pl.kernel(
    _tr_body,
    out_type=[jax.ShapeDtypeStruct((_N_MID, _D), jnp.float32),
              jax.ShapeDtypeStruct((_N_UID, _D), jnp.float32)],
    mesh=plsc.VectorSubcoreMesh(core_axis_name="c", subcore_axis_name="s",
                                num_cores=_NC, num_subcores=_NS),
    compiler_params=pltpu.CompilerParams(use_tc_tiling_on_sc=False,
                                         needs_layout_passes=False),
    scratch_types=[
        pltpu.VMEM((16 * 1040,), jnp.float32),
        pltpu.VMEM((1040, _D), jnp.float32),
        pltpu.VMEM((16 * 1040,), jnp.float32),
        pltpu.VMEM((1040, _D), jnp.float32),
        pltpu.SemaphoreType.DMA,
        pltpu.SemaphoreType.DMA,
        pltpu.SemaphoreType.DMA,
        pltpu.SemaphoreType.DMA,
    ],
)


def _sc_body(uid_emb, mid_emb, cat_emb, uid_b, mid_b, cat_b,
             mhis, chis, msk,
             uid_o, mid_o, cat_o, hm_o, hc_o,
             cat_tab, sidx, srows,
             hidx0, hcidx0, hmask0, hrows0,
             hidx1, hcidx1, hmask1, hrows1,
             hm_buf, hc_buf, tb_m, tb_c,
             sem, sg0, sg1, ss0, ss1):
    wid = lax.axis_index("c") * _NS + lax.axis_index("s")
    base = wid * _BPW

    hidx = (hidx0, hidx1)
    hcidx = (hcidx0, hcidx1)
    hmask = (hmask0, hmask1)
    hrows = (hrows0, hrows1)
    sg = (sg0, sg1)
    ss = (ss0, ss1)

    for p in range(2):
        hmask[p][pl.ds(_ROWS, 16)] = jnp.zeros((16,), jnp.float32)
        hcidx[p][pl.ds(_ROWS, 16)] = jnp.zeros((16,), jnp.int32)
        for r in range(8):
            hrows[p][_ROWS + r, :] = jnp.zeros((16,), jnp.float32)

    pltpu.sync_copy(cat_emb, cat_tab)

    for tab, idx_hbm, out_hbm in ((uid_emb, uid_b, uid_o),
                                  (mid_emb, mid_b, mid_o)):
        pltpu.sync_copy(idx_hbm.at[pl.ds(base, _BPW)], sidx)
        pltpu.async_copy(tab.at[sidx], srows, sem).wait()
        pltpu.sync_copy(srows, out_hbm.at[pl.ds(base, _BPW)])

    iota0 = lax.iota(jnp.int32, 16)
    dcon = [jnp.full((16,), d, jnp.int32) for d in range(16)]
    pltpu.sync_copy(cat_b.at[pl.ds(base, _BPW)], sidx)
    for k in range(_BPW // 16):
        cv = sidx[pl.ds(k * 16, 16)]
        rv = iota0 + k * 16
        for d in range(16):
            val = plsc.load_gather(cat_tab, [cv, dcon[d]])
            plsc.store_scatter(srows, [rv, dcon[d]], val)
    pltpu.sync_copy(srows, cat_o.at[pl.ds(base, _BPW)])

    def stage(p, c):
        fbase = base * _L + c * _ROWS
        pltpu.async_copy(mhis.at[pl.ds(fbase, _ROWS)],
                         hidx[p].at[pl.ds(0, _ROWS)], ss[p])
        pltpu.async_copy(chis.at[pl.ds(fbase, _ROWS)],
                         hcidx[p].at[pl.ds(0, _ROWS)], ss[p])
        pltpu.async_copy(msk.at[pl.ds(fbase, _ROWS)],
                         hmask[p].at[pl.ds(0, _ROWS)], ss[p])

    def drain_stage(p, c):
        fbase = base * _L + c * _ROWS
        pltpu.make_async_copy(mhis.at[pl.ds(fbase, _ROWS)],
                              hidx[p].at[pl.ds(0, _ROWS)], ss[p]).wait()
        pltpu.make_async_copy(chis.at[pl.ds(fbase, _ROWS)],
                              hcidx[p].at[pl.ds(0, _ROWS)], ss[p]).wait()
        pltpu.make_async_copy(msk.at[pl.ds(fbase, _ROWS)],
                              hmask[p].at[pl.ds(0, _ROWS)], ss[p]).wait()

    def fire(p):
        for g in range(12):
            pltpu.async_copy(mid_emb.at[hidx[p].at[pl.ds(g * 128, 128)]],
                             hrows[p].at[pl.ds(g * 128, 128)], sg[p])
        pltpu.async_copy(mid_emb.at[hidx[p].at[pl.ds(1536, 64)]],
                         hrows[p].at[pl.ds(1536, 64)], sg[p])

    def drain_fire(p):
        for g in range(12):
            pltpu.make_async_copy(mid_emb.at[hidx[p].at[pl.ds(g * 128, 128)]],
                                  hrows[p].at[pl.ds(g * 128, 128)],
                                  sg[p]).wait()
        pltpu.make_async_copy(mid_emb.at[hidx[p].at[pl.ds(1536, 64)]],
                              hrows[p].at[pl.ds(1536, 64)], sg[p]).wait()

    def compute(p, c):
        @pl.loop(0, _CHUNK_B)
        def _b(bi):
            iota = lax.iota(jnp.int32, 16)
            iota16 = iota * 16
            tailmask = (iota < 8).astype(jnp.float32)
            zf = jnp.zeros((16,), jnp.float32)
            dconsts = [jnp.full((16,), d, jnp.int32) for d in range(16)]
            rbase = bi * _L
            for half in range(2):
                accm = [zf] * 8
                accc = [zf] * 8
                for blk in range(_NBLK):
                    off = rbase + blk * 16
                    mv = hmask[p][pl.ds(off, 16)]
                    if blk == _NBLK - 1:
                        mv = mv * tailmask
                    cvv = hcidx[p][pl.ds(off, 16)]
                    rv = iota + off
                    for dd in range(8):
                        d = half * 8 + dd
                        mrow = plsc.load_gather(hrows[p], [rv, dconsts[d]])
                        crow = plsc.load_gather(cat_tab, [cvv, dconsts[d]])
                        accm[dd] = accm[dd] + mv * mrow
                        accc[dd] = accc[dd] + mv * crow
                for dd in range(8):
                    d = half * 8 + dd
                    tb_m[pl.ds(d * 16, 16)] = accm[dd]
                    tb_c[pl.ds(d * 16, 16)] = accc[dd]
            rm = zf
            rc = zf
            for k in range(16):
                rm = rm + plsc.load_gather(tb_m, [iota16 + k])
                rc = rc + plsc.load_gather(tb_c, [iota16 + k])
            hm_buf[c * _CHUNK_B + bi, :] = rm
            hc_buf[c * _CHUNK_B + bi, :] = rc

    stage(0, 0)
    drain_stage(0, 0)
    fire(0)
    stage(1, 1)

    @pl.loop(0, _N_CHUNK, step=2)
    def _chunks(c0):
        for ph in range(2):
            p = ph
            c = c0 + ph
            drain_fire(p)

            @pl.when(c + 1 < _N_CHUNK)
            def _():
                drain_stage(1 - p, c + 1)
                fire(1 - p)

            compute(p, c)

            @pl.when(c + 2 < _N_CHUNK)
            def _():
                stage(p, c + 2)

    pltpu.sync_copy(hm_buf, hm_o.at[pl.ds(base, _BPW)])
    pltpu.sync_copy(hc_buf, hc_o.at[pl.ds(base, _BPW)])


_sc_embed = pl.kernel(
    _sc_body,
    out_type=[jax.ShapeDtypeStruct((_B, _D), jnp.float32)] * 5,
    mesh=plsc.VectorSubcoreMesh(core_axis_name="c", subcore_axis_name="s",
                                num_cores=_NC, num_subcores=_NS),
    compiler_params=pltpu.CompilerParams(use_tc_tiling_on_sc=False,
                                         needs_layout_passes=False),
    scratch_types=[
        pltpu.VMEM((1000, _D), jnp.float32),
        pltpu.VMEM((_BPW,), jnp.int32),
        pltpu.VMEM((_BPW, _D), jnp.float32),
        pltpu.VMEM((_ROWS + 16,), jnp.int32),
        pltpu.VMEM((_ROWS + 16,), jnp.int32),
        pltpu.VMEM((_ROWS + 16,), jnp.float32),
        pltpu.VMEM((_PAD_ROWS, _D), jnp.float32),
        pltpu.VMEM((_ROWS + 16,), jnp.int32),
        pltpu.VMEM((_ROWS + 16,), jnp.int32),
        pltpu.VMEM((_ROWS + 16,), jnp.float32),
        pltpu.VMEM((_PAD_ROWS, _D), jnp.float32),
        pltpu.VMEM((_BPW, _D), jnp.float32),
        pltpu.VMEM((_BPW, _D), jnp.float32),
        pltpu.VMEM((256,), jnp.float32),
        pltpu.VMEM((256,), jnp.float32),
        pltpu.SemaphoreType.DMA,
        pltpu.SemaphoreType.DMA,
        pltpu.SemaphoreType.DMA,
        pltpu.SemaphoreType.DMA,
        pltpu.SemaphoreType.DMA,
    ],
)


def _prelu(x, alpha):
    return jnp.maximum(0.0, x) + alpha * jnp.minimum(0.0, x)


def _tc_body(uid_e, mid_e, cat_e, hm, hc, gamma, beta, W1, b1, a1, W2, b2,
             a2, W3, b3, Ww, bw, out_ref):
    uid = uid_e[...]
    mid = mid_e[...]
    cat = cat_e[...]
    hms = hm[...]
    hcs = hc[...]
    inp = jnp.concatenate([uid, mid, cat, hms, hcs], axis=1)
    mu = jnp.mean(inp, axis=0, keepdims=True)
    var = jnp.mean((inp - mu) ** 2, axis=0, keepdims=True)
    bn = gamma[...] * (inp - mu) / jnp.sqrt(var + 1e-3) + beta[...]
    d1 = _prelu(jnp.dot(bn, W1[...], preferred_element_type=jnp.float32)
                + b1[...], a1[...])
    d2 = _prelu(jnp.dot(d1, W2[...], preferred_element_type=jnp.float32)
                + b2[...], a2[...])
    d3 = jnp.dot(d2, W3[...], preferred_element_type=jnp.float32) + b3[...]
    item = jnp.concatenate([mid, cat], axis=1)
    hsum = jnp.concatenate([hms, hcs], axis=1)
    wide_in = jnp.concatenate([item, hsum, item * hsum], axis=1)
    wide = jnp.dot(wide_in, Ww[...], preferred_element_type=jnp.float32) + bw[...]
    logits = d3 + wide
    mx = jnp.max(logits, axis=-1, keepdims=True)
    e = jnp.exp(logits - mx)
    out_ref[...] = e / jnp.sum(e, axis=-1, keepdims=True) + 1e-8


_tc_head = pl.pallas_call(
    _tc_body,
    out_shape=jax.ShapeDtypeStruct((_B, 2), jnp.float32),
)


def kernel(uid_emb, mid_emb, cat_emb, gamma, beta, W1, b1, alpha1, W2, b2,
           alpha2, W3, b3, Ww, bw, mask, uid_batch, mid_batch, cat_batch,
           mid_his_batch, cat_his_batch):
    mhis = mid_his_batch.reshape(-1)
    chis = cat_his_batch.reshape(-1)
    msk = mask.reshape(-1)
    mid_rm, uid_rm = _sc_transpose(mid_emb.T.reshape(-1),
                                   uid_emb.T.reshape(-1))
    uid_e, mid_e, cat_e, hm, hc = _sc_embed(
        uid_rm, mid_rm, cat_emb, uid_batch, mid_batch, cat_batch,
        mhis, chis, msk)
    return _tc_head(uid_e, mid_e, cat_e, hm, hc, gamma.reshape(1, -1),
                    beta.reshape(1, -1), W1, b1.reshape(1, -1),
                    alpha1.reshape(1, -1), W2, b2.reshape(1, -1),
                    alpha2.reshape(1, -1), W3, b3.reshape(1, -1), Ww,
                    bw.reshape(1, -1))


# speedup vs baseline: 2.3657x; 2.3657x over previous
import jax
import jax.numpy as jnp
from jax import lax
from jax.experimental import pallas as pl
from jax.experimental.pallas import tpu as pltpu
from jax.experimental.pallas import tpu_sc as plsc

_B, _L, _D = 4096, 200, 16
_NC, _NS = 2, 16
_NW = _NC * _NS
_BPW = _B // _NW
_CHUNK_B = 8
_N_CHUNK = _BPW // _CHUNK_B
_ROWS = _CHUNK_B * _L
_PAD_ROWS = _ROWS + 8
_NBLK = 13


def _sc_body(uid_emb, mid_emb, cat_emb, uid_b, mid_b, cat_b,
             mhis, chis, msk,
             uid_o, mid_o, cat_o, hm_o, hc_o,
             cat_tab, sidx, srows,
             hidx0, hcidx0, hmask0, hrows0,
             hidx1, hcidx1, hmask1, hrows1,
             hm_buf, hc_buf, tb_m, tb_c,
             sem, sg0, sg1, ss0, ss1):
    wid = lax.axis_index("c") * _NS + lax.axis_index("s")
    base = wid * _BPW

    hidx = (hidx0, hidx1)
    hcidx = (hcidx0, hcidx1)
    hmask = (hmask0, hmask1)
    hrows = (hrows0, hrows1)
    sg = (sg0, sg1)
    ss = (ss0, ss1)

    for p in range(2):
        hmask[p][pl.ds(_ROWS, 16)] = jnp.zeros((16,), jnp.float32)
        hcidx[p][pl.ds(_ROWS, 16)] = jnp.zeros((16,), jnp.int32)
        for r in range(8):
            hrows[p][_ROWS + r, :] = jnp.zeros((16,), jnp.float32)

    pltpu.sync_copy(cat_emb, cat_tab)

    for tab, idx_hbm, out_hbm in ((uid_emb, uid_b, uid_o),
                                  (mid_emb, mid_b, mid_o)):
        pltpu.sync_copy(idx_hbm.at[pl.ds(base, _BPW)], sidx)
        pltpu.async_copy(tab.at[sidx], srows, sem).wait()
        pltpu.sync_copy(srows, out_hbm.at[pl.ds(base, _BPW)])

    iota0 = lax.iota(jnp.int32, 16)
    dcon = [jnp.full((16,), d, jnp.int32) for d in range(16)]
    pltpu.sync_copy(cat_b.at[pl.ds(base, _BPW)], sidx)
    for k in range(_BPW // 16):
        cv = sidx[pl.ds(k * 16, 16)]
        rv = iota0 + k * 16
        for d in range(16):
            val = plsc.load_gather(cat_tab, [cv, dcon[d]])
            plsc.store_scatter(srows, [rv, dcon[d]], val)
    pltpu.sync_copy(srows, cat_o.at[pl.ds(base, _BPW)])

    def stage(p, c):
        fbase = base * _L + c * _ROWS
        pltpu.async_copy(mhis.at[pl.ds(fbase, _ROWS)],
                         hidx[p].at[pl.ds(0, _ROWS)], ss[p])
        pltpu.async_copy(chis.at[pl.ds(fbase, _ROWS)],
                         hcidx[p].at[pl.ds(0, _ROWS)], ss[p])
        pltpu.async_copy(msk.at[pl.ds(fbase, _ROWS)],
                         hmask[p].at[pl.ds(0, _ROWS)], ss[p])

    def drain_stage(p, c):
        fbase = base * _L + c * _ROWS
        pltpu.make_async_copy(mhis.at[pl.ds(fbase, _ROWS)],
                              hidx[p].at[pl.ds(0, _ROWS)], ss[p]).wait()
        pltpu.make_async_copy(chis.at[pl.ds(fbase, _ROWS)],
                              hcidx[p].at[pl.ds(0, _ROWS)], ss[p]).wait()
        pltpu.make_async_copy(msk.at[pl.ds(fbase, _ROWS)],
                              hmask[p].at[pl.ds(0, _ROWS)], ss[p]).wait()

    def fire(p):
        for g in range(12):
            pltpu.async_copy(mid_emb.at[hidx[p].at[pl.ds(g * 128, 128)]],
                             hrows[p].at[pl.ds(g * 128, 128)], sg[p])
        pltpu.async_copy(mid_emb.at[hidx[p].at[pl.ds(1536, 64)]],
                         hrows[p].at[pl.ds(1536, 64)], sg[p])

    def drain_fire(p):
        for g in range(12):
            pltpu.make_async_copy(mid_emb.at[hidx[p].at[pl.ds(g * 128, 128)]],
                                  hrows[p].at[pl.ds(g * 128, 128)],
                                  sg[p]).wait()
        pltpu.make_async_copy(mid_emb.at[hidx[p].at[pl.ds(1536, 64)]],
                              hrows[p].at[pl.ds(1536, 64)], sg[p]).wait()

    def compute(p, c):
        @pl.loop(0, _CHUNK_B)
        def _b(bi):
            iota = lax.iota(jnp.int32, 16)
            iota16 = iota * 16
            tailmask = (iota < 8).astype(jnp.float32)
            zf = jnp.zeros((16,), jnp.float32)
            dconsts = [jnp.full((16,), d, jnp.int32) for d in range(16)]
            rbase = bi * _L
            for half in range(2):
                accm = [zf] * 8
                accc = [zf] * 8
                for blk in range(_NBLK):
                    off = rbase + blk * 16
                    mv = hmask[p][pl.ds(off, 16)]
                    if blk == _NBLK - 1:
                        mv = mv * tailmask
                    cvv = hcidx[p][pl.ds(off, 16)]
                    rv = iota + off
                    for dd in range(8):
                        d = half * 8 + dd
                        mrow = plsc.load_gather(hrows[p], [rv, dconsts[d]])
                        crow = plsc.load_gather(cat_tab, [cvv, dconsts[d]])
                        accm[dd] = accm[dd] + mv * mrow
                        accc[dd] = accc[dd] + mv * crow
                for dd in range(8):
                    d = half * 8 + dd
                    tb_m[pl.ds(d * 16, 16)] = accm[dd]
                    tb_c[pl.ds(d * 16, 16)] = accc[dd]
            rm = zf
            rc = zf
            for k in range(16):
                rm = rm + plsc.load_gather(tb_m, [iota16 + k])
                rc = rc + plsc.load_gather(tb_c, [iota16 + k])
            hm_buf[c * _CHUNK_B + bi, :] = rm
            hc_buf[c * _CHUNK_B + bi, :] = rc

    stage(0, 0)
    drain_stage(0, 0)
    fire(0)
    stage(1, 1)

    @pl.loop(0, _N_CHUNK, step=2)
    def _chunks(c0):
        for ph in range(2):
            p = ph
            c = c0 + ph
            drain_fire(p)

            @pl.when(c + 1 < _N_CHUNK)
            def _():
                drain_stage(1 - p, c + 1)
                fire(1 - p)

            compute(p, c)

            @pl.when(c + 2 < _N_CHUNK)
            def _():
                stage(p, c + 2)

    pltpu.sync_copy(hm_buf, hm_o.at[pl.ds(base, _BPW)])
    pltpu.sync_copy(hc_buf, hc_o.at[pl.ds(base, _BPW)])


_sc_embed = pl.kernel(
    _sc_body,
    out_type=[jax.ShapeDtypeStruct((_B, _D), jnp.float32)] * 5,
    mesh=plsc.VectorSubcoreMesh(core_axis_name="c", subcore_axis_name="s",
                                num_cores=_NC, num_subcores=_NS),
    compiler_params=pltpu.CompilerParams(use_tc_tiling_on_sc=False,
                                         needs_layout_passes=False),
    scratch_types=[
        pltpu.VMEM((1000, _D), jnp.float32),
        pltpu.VMEM((_BPW,), jnp.int32),
        pltpu.VMEM((_BPW, _D), jnp.float32),
        pltpu.VMEM((_ROWS + 16,), jnp.int32),
        pltpu.VMEM((_ROWS + 16,), jnp.int32),
        pltpu.VMEM((_ROWS + 16,), jnp.float32),
        pltpu.VMEM((_PAD_ROWS, _D), jnp.float32),
        pltpu.VMEM((_ROWS + 16,), jnp.int32),
        pltpu.VMEM((_ROWS + 16,), jnp.int32),
        pltpu.VMEM((_ROWS + 16,), jnp.float32),
        pltpu.VMEM((_PAD_ROWS, _D), jnp.float32),
        pltpu.VMEM((_BPW, _D), jnp.float32),
        pltpu.VMEM((_BPW, _D), jnp.float32),
        pltpu.VMEM((256,), jnp.float32),
        pltpu.VMEM((256,), jnp.float32),
        pltpu.SemaphoreType.DMA,
        pltpu.SemaphoreType.DMA,
        pltpu.SemaphoreType.DMA,
        pltpu.SemaphoreType.DMA,
        pltpu.SemaphoreType.DMA,
    ],
)


def _prelu(x, alpha):
    return jnp.maximum(0.0, x) + alpha * jnp.minimum(0.0, x)


def _tc_body(uid_e, mid_e, cat_e, hm, hc, gamma, beta, W1, b1, a1, W2, b2,
             a2, W3, b3, Ww, bw, out_ref):
    uid = uid_e[...]
    mid = mid_e[...]
    cat = cat_e[...]
    hms = hm[...]
    hcs = hc[...]
    inp = jnp.concatenate([uid, mid, cat, hms, hcs], axis=1)
    mu = jnp.mean(inp, axis=0, keepdims=True)
    var = jnp.mean((inp - mu) ** 2, axis=0, keepdims=True)
    bn = gamma[...] * (inp - mu) / jnp.sqrt(var + 1e-3) + beta[...]
    d1 = _prelu(jnp.dot(bn, W1[...], preferred_element_type=jnp.float32)
                + b1[...], a1[...])
    d2 = _prelu(jnp.dot(d1, W2[...], preferred_element_type=jnp.float32)
                + b2[...], a2[...])
    d3 = jnp.dot(d2, W3[...], preferred_element_type=jnp.float32) + b3[...]
    item = jnp.concatenate([mid, cat], axis=1)
    hsum = jnp.concatenate([hms, hcs], axis=1)
    wide_in = jnp.concatenate([item, hsum, item * hsum], axis=1)
    wide = jnp.dot(wide_in, Ww[...], preferred_element_type=jnp.float32) + bw[...]
    logits = d3 + wide
    mx = jnp.max(logits, axis=-1, keepdims=True)
    e = jnp.exp(logits - mx)
    out_ref[...] = e / jnp.sum(e, axis=-1, keepdims=True) + 1e-8


_tc_head = pl.pallas_call(
    _tc_body,
    out_shape=jax.ShapeDtypeStruct((_B, 2), jnp.float32),
)


def kernel(uid_emb, mid_emb, cat_emb, gamma, beta, W1, b1, alpha1, W2, b2,
           alpha2, W3, b3, Ww, bw, mask, uid_batch, mid_batch, cat_batch,
           mid_his_batch, cat_his_batch):
    mhis = mid_his_batch.reshape(-1)
    chis = cat_his_batch.reshape(-1)
    msk = mask.reshape(-1)
    uid_e, mid_e, cat_e, hm, hc = _sc_embed(
        uid_emb, mid_emb, cat_emb, uid_batch, mid_batch, cat_batch,
        mhis, chis, msk)
    return _tc_head(uid_e, mid_e, cat_e, hm, hc, gamma.reshape(1, -1),
                    beta.reshape(1, -1), W1, b1.reshape(1, -1),
                    alpha1.reshape(1, -1), W2, b2.reshape(1, -1),
                    alpha2.reshape(1, -1), W3, b3.reshape(1, -1), Ww,
                    bw.reshape(1, -1))
